# baseline (speedup 1.0000x reference)
import functools

import jax
import jax.numpy as jnp
import numpy as np
from jax import lax
from jax.experimental import pallas as pl
from jax.experimental.pallas import tpu as pltpu
from jax.experimental.pallas import tpu_sc as plsc

EPS = 1e-07
NT = 200
N_TOTAL = 4194304
NC, NS, LANES = 2, 16, 16
NW = NC * NS
EPT = N_TOTAL // NW
CH = 16384
NCHUNK = EPT // CH
BINS = 512
ACCN = LANES * BINS


def _thr_table():
    thr = [(i + 1) * 1.0 / (NT - 1) for i in range(NT - 2)]
    thr = [0.0 - EPS] + thr + [1.0 + EPS] + [2.0] * 8
    return jnp.asarray(np.repeat(np.asarray(thr, np.float32), LANES))


def _sc_hist(preds, targets, thr):
    mesh = plsc.VectorSubcoreMesh(core_axis_name="c", subcore_axis_name="s")

    @functools.partial(
        pl.kernel,
        out_type=jax.ShapeDtypeStruct((NW, ACCN), jnp.float32),
        mesh=mesh,
        compiler_params=pltpu.CompilerParams(needs_layout_passes=False),
        scratch_types=[
            pltpu.VMEM((CH,), jnp.float32),
            pltpu.VMEM((CH,), jnp.float32),
            pltpu.VMEM((CH,), jnp.float32),
            pltpu.VMEM((CH,), jnp.float32),
            pltpu.VMEM((208 * LANES,), jnp.float32),
            pltpu.VMEM((ACCN,), jnp.float32),
            pltpu.SemaphoreType.DMA,
            pltpu.SemaphoreType.DMA,
            pltpu.SemaphoreType.DMA,
            pltpu.SemaphoreType.DMA,
        ],
    )
    def k(preds_hbm, targets_hbm, thr_hbm, out_hbm,
          pbuf0, tbuf0, pbuf1, tbuf1, thrv, acc, sp0, st0, sp1, st1):
        wid = lax.axis_index("s") * NC + lax.axis_index("c")
        base = wid * EPT

        def start_even(c):
            off = base + c * CH
            pltpu.async_copy(preds_hbm.at[pl.ds(off, CH)], pbuf0, sp0)
            pltpu.async_copy(targets_hbm.at[pl.ds(off, CH)], tbuf0, st0)

        def start_odd(c):
            off = base + c * CH
            pltpu.async_copy(preds_hbm.at[pl.ds(off, CH)], pbuf1, sp1)
            pltpu.async_copy(targets_hbm.at[pl.ds(off, CH)], tbuf1, st1)

        start_even(0)
        pltpu.sync_copy(thr_hbm, thrv)

        zeros16 = jnp.zeros((LANES,), jnp.float32)

        @plsc.parallel_loop(0, ACCN, step=LANES, unroll=8)
        def _zero(i):
            acc[pl.ds(i, LANES)] = zeros16

        ones16 = jnp.full((LANES,), 1.0, jnp.float32)
        lane_iota = lax.iota(jnp.int32, 16)
        magic = jnp.float32(12582912.0)
        gbase = lane_iota + jnp.full((LANES,), 1275068416, jnp.int32)

        def process(pb, tb):
            @plsc.parallel_loop(0, CH, step=LANES, unroll=8)
            def _body(i):
                s = pl.ds(i, LANES)
                p = pb[s]
                v = p * 199.0 + magic
                gidx = (plsc.bitcast(v, jnp.int32) << 4) + gbase
                tv = plsc.load_gather(thrv, [gidx])
                t = tb[s]
                idx = (gidx
                       + jnp.where(tv < p, 16, 0)
                       + jnp.where(t == 1.0, 4096, 0))
                plsc.addupdate_scatter(acc, [idx], ones16)

        @pl.loop(0, NCHUNK, step=2)
        def _chunks(c):
            start_odd(c + 1)
            pltpu.make_async_copy(preds_hbm.at[pl.ds(0, CH)], pbuf0, sp0).wait()
            pltpu.make_async_copy(targets_hbm.at[pl.ds(0, CH)], tbuf0, st0).wait()
            process(pbuf0, tbuf0)

            @pl.when(c + 2 < NCHUNK)
            def _():
                start_even(c + 2)

            pltpu.make_async_copy(preds_hbm.at[pl.ds(0, CH)], pbuf1, sp1).wait()
            pltpu.make_async_copy(targets_hbm.at[pl.ds(0, CH)], tbuf1, st1).wait()
            process(pbuf1, tbuf1)

        pltpu.sync_copy(acc, out_hbm.at[wid])

    return k(preds, targets, thr)


def _tc_auc(partials):
    def body(h_ref, o_ref):
        dot = functools.partial(lax.dot, precision=lax.Precision.HIGHEST)
        x = h_ref[...]
        s1 = dot(jnp.ones((1, NW), jnp.float32), x)
        ki = lax.broadcasted_iota(jnp.int32, (4096, 256), 0) // 16
        kj = lax.broadcasted_iota(jnp.int32, (4096, 256), 1)
        k256 = (ki == kj).astype(jnp.float32)
        h_nt = dot(s1[:, 0:4096], k256)
        h_t = dot(s1[:, 4096:8192], k256)
        r = lax.broadcasted_iota(jnp.int32, (256, 512), 0)
        cc = lax.broadcasted_iota(jnp.int32, (256, 512), 1)
        ccm = cc % 256
        m_both = jnp.where(cc < 256, (r <= ccm).astype(jnp.float32),
                           (jnp.maximum(r - 1, 0) >= ccm).astype(jnp.float32))
        nt_cr = dot(h_nt, m_both)
        t_cr = dot(h_t, m_both)
        tn = nt_cr[:, 0:256]
        fp = nt_cr[:, 256:512]
        fn = t_cr[:, 0:256]
        tp = t_cr[:, 256:512]
        x = fp / (fp + tn + EPS)
        y = (tp + EPS) / (tp + fn + EPS)
        xy = jnp.concatenate([x, y], axis=0)
        r2 = lax.broadcasted_iota(jnp.int32, (256, 256), 0)
        c2 = lax.broadcasted_iota(jnp.int32, (256, 256), 1)
        shift = (r2 == c2 + 1).astype(jnp.float32)
        xys = dot(xy, shift)
        xs = xys[0:1, :]
        ys = xys[1:2, :]
        j = lax.broadcasted_iota(jnp.int32, (1, 256), 1)
        terms = jnp.where(j <= NT - 2, (x - xs) * (y + ys) * 0.5, 0.0)
        o_ref[...] = jnp.sum(terms, axis=1, keepdims=True)

    return pl.pallas_call(
        body,
        out_shape=jax.ShapeDtypeStruct((1, 1), jnp.float32),
    )(partials)


def kernel(preds, targets):
    p = preds.reshape(-1)
    t = targets.reshape(-1)
    hist = _sc_hist(p, t, _thr_table())
    roc = _tc_auc(hist)
    return roc.reshape(())


# speedup vs baseline: 1.0304x; 1.0304x over previous
import functools

import jax
import jax.numpy as jnp
import numpy as np
from jax import lax
from jax.experimental import pallas as pl
from jax.experimental.pallas import tpu as pltpu
from jax.experimental.pallas import tpu_sc as plsc

EPS = 1e-07
NT = 200
N_TOTAL = 4194304
NC, NS, LANES = 2, 16, 16
NW = NC * NS
EPT = N_TOTAL // NW
CH = 16384
NCHUNK = EPT // CH
BINS = 512
ACCN = LANES * BINS


def _thr_table():
    thr = [(i + 1) * 1.0 / (NT - 1) for i in range(NT - 2)]
    thr = [0.0 - EPS] + thr + [1.0 + EPS] + [2.0] * 8
    return jnp.asarray(np.repeat(np.asarray(thr, np.float32), LANES))


def _sc_hist(preds, targets, thr):
    mesh = plsc.VectorSubcoreMesh(core_axis_name="c", subcore_axis_name="s")

    @functools.partial(
        pl.kernel,
        out_type=jax.ShapeDtypeStruct((NW, ACCN), jnp.float32),
        mesh=mesh,
        compiler_params=pltpu.CompilerParams(needs_layout_passes=False),
        scratch_types=[
            pltpu.VMEM((CH,), jnp.float32),
            pltpu.VMEM((CH,), jnp.float32),
            pltpu.VMEM((CH,), jnp.float32),
            pltpu.VMEM((CH,), jnp.float32),
            pltpu.VMEM((208 * LANES,), jnp.float32),
            pltpu.VMEM((ACCN,), jnp.float32),
            pltpu.SemaphoreType.DMA,
            pltpu.SemaphoreType.DMA,
            pltpu.SemaphoreType.DMA,
            pltpu.SemaphoreType.DMA,
        ],
    )
    def k(preds_hbm, targets_hbm, thr_hbm, out_hbm,
          pbuf0, tbuf0, pbuf1, tbuf1, thrv, acc, sp0, st0, sp1, st1):
        wid = lax.axis_index("s") * NC + lax.axis_index("c")
        base = wid * EPT

        def start_even(c):
            off = base + c * CH
            pltpu.async_copy(preds_hbm.at[pl.ds(off, CH)], pbuf0, sp0)
            pltpu.async_copy(targets_hbm.at[pl.ds(off, CH)], tbuf0, st0)

        def start_odd(c):
            off = base + c * CH
            pltpu.async_copy(preds_hbm.at[pl.ds(off, CH)], pbuf1, sp1)
            pltpu.async_copy(targets_hbm.at[pl.ds(off, CH)], tbuf1, st1)

        start_even(0)
        pltpu.sync_copy(thr_hbm, thrv)

        zeros16 = jnp.zeros((LANES,), jnp.float32)

        @plsc.parallel_loop(0, ACCN, step=LANES, unroll=8)
        def _zero(i):
            acc[pl.ds(i, LANES)] = zeros16

        ones16 = jnp.full((LANES,), 1.0, jnp.float32)
        lane_iota = lax.iota(jnp.int32, 16)
        magic = jnp.float32(12582912.0)
        gbase = lane_iota + jnp.full((LANES,), 1275068416, jnp.int32)

        def process(pb, tb):
            @plsc.parallel_loop(0, CH, step=LANES, unroll=8)
            def _body(i):
                s = pl.ds(i, LANES)
                p = pb[s]
                v = p * 199.0 + magic
                gidx = (plsc.bitcast(v, jnp.int32) << 4) + gbase
                tv = plsc.load_gather(thrv, [gidx])
                t = tb[s]
                idx = (gidx
                       + jnp.where(tv < p, 16, 0)
                       + jnp.where(t == 1.0, 4096, 0))
                plsc.addupdate_scatter(acc, [idx], ones16)

        @pl.loop(0, NCHUNK, step=2)
        def _chunks(c):
            start_odd(c + 1)
            pltpu.make_async_copy(preds_hbm.at[pl.ds(0, CH)], pbuf0, sp0).wait()
            pltpu.make_async_copy(targets_hbm.at[pl.ds(0, CH)], tbuf0, st0).wait()
            process(pbuf0, tbuf0)

            @pl.when(c + 2 < NCHUNK)
            def _():
                start_even(c + 2)

            pltpu.make_async_copy(preds_hbm.at[pl.ds(0, CH)], pbuf1, sp1).wait()
            pltpu.make_async_copy(targets_hbm.at[pl.ds(0, CH)], tbuf1, st1).wait()
            process(pbuf1, tbuf1)

        pltpu.sync_copy(acc, out_hbm.at[wid])

    return k(preds, targets, thr)


def _tc_auc(partials):
    def body(h_ref, o_ref):
        dot = functools.partial(lax.dot, precision=lax.Precision.HIGHEST)
        x = h_ref[...]
        s1 = dot(jnp.ones((1, NW), jnp.float32), x)
        ki = lax.broadcasted_iota(jnp.int32, (1024, 64), 0) // 16
        kj = lax.broadcasted_iota(jnp.int32, (1024, 64), 1)
        k64 = (ki == kj).astype(jnp.float32)
        h_all = jnp.concatenate(
            [dot(s1[:, b * 1024:(b + 1) * 1024], k64) for b in range(8)],
            axis=1)
        h_nt = h_all[:, 0:256]
        h_t = h_all[:, 256:512]
        r = lax.broadcasted_iota(jnp.int32, (256, 512), 0)
        cc = lax.broadcasted_iota(jnp.int32, (256, 512), 1)
        ccm = cc % 256
        m_both = jnp.where(cc < 256, (r <= ccm).astype(jnp.float32),
                           (jnp.maximum(r - 1, 0) >= ccm).astype(jnp.float32))
        nt_cr = dot(h_nt, m_both)
        t_cr = dot(h_t, m_both)
        tn = nt_cr[:, 0:256]
        fp = nt_cr[:, 256:512]
        fn = t_cr[:, 0:256]
        tp = t_cr[:, 256:512]
        x = fp / (fp + tn + EPS)
        y = (tp + EPS) / (tp + fn + EPS)
        xy = jnp.concatenate([x, y], axis=0)
        r2 = lax.broadcasted_iota(jnp.int32, (256, 256), 0)
        c2 = lax.broadcasted_iota(jnp.int32, (256, 256), 1)
        shift = (r2 == c2 + 1).astype(jnp.float32)
        xys = dot(xy, shift)
        xs = xys[0:1, :]
        ys = xys[1:2, :]
        j = lax.broadcasted_iota(jnp.int32, (1, 256), 1)
        terms = jnp.where(j <= NT - 2, (x - xs) * (y + ys) * 0.5, 0.0)
        o_ref[...] = jnp.sum(terms, axis=1, keepdims=True)

    return pl.pallas_call(
        body,
        out_shape=jax.ShapeDtypeStruct((1, 1), jnp.float32),
    )(partials)


def kernel(preds, targets):
    p = preds.reshape(-1)
    t = targets.reshape(-1)
    hist = _sc_hist(p, t, _thr_table())
    roc = _tc_auc(hist)
    return roc.reshape(())
